# baseline (speedup 1.0000x reference)
import jax
import jax.numpy as jnp
from jax.experimental import pallas as pl
from jax.experimental.pallas import tpu as pltpu


def _passthrough(img_ref, lab_ref, img_out, lab_out):
    img_out[...] = img_ref[...]

    @pl.when(pl.program_id(0) == 0)
    def _():
        lab_out[...] = lab_ref[...]


def kernel(images, labels):
    B, H, W, C = images.shape
    img2 = images.transpose(0, 3, 1, 2).reshape(B * C * H, W)
    rows, cols = img2.shape
    grid = 6
    blk = rows // grid
    out_img, out_lab = pl.pallas_call(
        _passthrough,
        grid=(grid,),
        in_specs=[
            pl.BlockSpec((blk, cols), lambda i: (i, 0)),
            pl.BlockSpec(labels.shape, lambda i: (0, 0)),
        ],
        out_specs=[
            pl.BlockSpec((blk, cols), lambda i: (i, 0)),
            pl.BlockSpec(labels.shape, lambda i: (0, 0)),
        ],
        out_shape=[
            jax.ShapeDtypeStruct(img2.shape, img2.dtype),
            jax.ShapeDtypeStruct(labels.shape, labels.dtype),
        ],
    )(img2, labels)
    return (out_img.reshape(B, C, H, W).transpose(0, 2, 3, 1), out_lab)


# speedup vs baseline: 1.0895x; 1.0895x over previous
import jax
import jax.numpy as jnp
from jax.experimental import pallas as pl
from jax.experimental.pallas import tpu as pltpu


def _passthrough(img_ref, lab_ref, img_out, lab_out):
    img_out[...] = img_ref[...]

    @pl.when(pl.program_id(0) == 0)
    def _():
        lab_out[...] = lab_ref[...]


def kernel(images, labels):
    B, H, W, C = images.shape
    img2 = images.transpose(0, 3, 1, 2).reshape(B * C * H, W)
    rows, cols = img2.shape
    grid = 3
    blk = rows // grid
    out_img, out_lab = pl.pallas_call(
        _passthrough,
        grid=(grid,),
        in_specs=[
            pl.BlockSpec((blk, cols), lambda i: (i, 0)),
            pl.BlockSpec(labels.shape, lambda i: (0, 0)),
        ],
        out_specs=[
            pl.BlockSpec((blk, cols), lambda i: (i, 0)),
            pl.BlockSpec(labels.shape, lambda i: (0, 0)),
        ],
        out_shape=[
            jax.ShapeDtypeStruct(img2.shape, img2.dtype),
            jax.ShapeDtypeStruct(labels.shape, labels.dtype),
        ],
    )(img2, labels)
    return (out_img.reshape(B, C, H, W).transpose(0, 2, 3, 1), out_lab)


# speedup vs baseline: 1.0956x; 1.0057x over previous
import jax
import jax.numpy as jnp
from jax.experimental import pallas as pl
from jax.experimental.pallas import tpu as pltpu


def _passthrough(img_ref, lab_ref, img_out, lab_out):
    img_out[...] = img_ref[...]

    @pl.when(pl.program_id(0) == 0)
    def _():
        lab_out[...] = lab_ref[...]


def kernel(images, labels):
    B, H, W, C = images.shape
    img2 = images.transpose(0, 3, 1, 2).reshape(B * C * H, W)
    rows, cols = img2.shape
    grid = 2
    blk = rows // grid
    out_img, out_lab = pl.pallas_call(
        _passthrough,
        grid=(grid,),
        in_specs=[
            pl.BlockSpec((blk, cols), lambda i: (i, 0)),
            pl.BlockSpec(labels.shape, lambda i: (0, 0)),
        ],
        out_specs=[
            pl.BlockSpec((blk, cols), lambda i: (i, 0)),
            pl.BlockSpec(labels.shape, lambda i: (0, 0)),
        ],
        out_shape=[
            jax.ShapeDtypeStruct(img2.shape, img2.dtype),
            jax.ShapeDtypeStruct(labels.shape, labels.dtype),
        ],
    )(img2, labels)
    return (out_img.reshape(B, C, H, W).transpose(0, 2, 3, 1), out_lab)
